# baseline (speedup 1.0000x reference)
import functools

import jax
import jax.numpy as jnp
from jax import lax
from jax.experimental import pallas as pl
from jax.experimental.pallas import tpu as pltpu
from jax.experimental.pallas import tpu_sc as plsc

N = 10000
E = 320000
D = 128
H = 128
O = 128

NC = 2
NS = 16
NW = NC * NS
EPW = E // NW
K = 80
STEPS = EPW // K
RPT = 624
TAIL = N - RPT * NS


def _sc_body(node_hbm, ef_hbm, sd3_hbm, zeros_hbm, part_hbm,
             ibuf, rows, ef, acc, gsem, esem, ssem, s2sem, isem):
    c = lax.axis_index("c")
    s = lax.axis_index("s")
    wid = c * NS + s
    base = wid * EPW

    pltpu.sync_copy(zeros_hbm.at[pl.ds(s * RPT, RPT)],
                    acc.at[pl.ds(s * RPT, RPT)])
    @pl.when(s == NS - 1)
    def _():
        pltpu.sync_copy(zeros_hbm.at[pl.ds(RPT * NS, TAIL)],
                        acc.at[pl.ds(RPT * NS, TAIL)])
    plsc.subcore_barrier()

    def idx_start(i, slot):
        pltpu.async_copy(sd3_hbm.at[wid, i], ibuf.at[slot], isem)

    def idx_wait(i, slot):
        pltpu.make_async_copy(sd3_hbm.at[wid, i], ibuf.at[slot], isem).wait()

    def gather_desc(i, b, slot):
        return pltpu.make_async_copy(
            node_hbm.at[ibuf.at[slot, 0]], rows.at[b], gsem)

    def ef_desc(i, b):
        return pltpu.make_async_copy(
            ef_hbm.at[pl.ds(base + i * K, K)], ef.at[b], esem)

    def scat_desc(i, b, slot):
        return pltpu.make_async_copy(
            rows.at[b], acc.at[ibuf.at[slot, 1]], ssem)

    def scat_ef_desc(i, b, slot):
        return pltpu.make_async_copy(
            ef.at[b], acc.at[ibuf.at[slot, 1]], s2sem)

    def step(i, b, slot, first=False, last=False):
        nb = 1 - b
        nslot = (slot + 1) % 4
        if not last:
            idx_start(i + 1, nslot)
        gather_desc(i, b, slot).wait()
        ef_desc(i, b).wait()
        if not first:
            scat_desc(i - 1, nb, (slot + 3) % 4).wait()
            scat_ef_desc(i - 1, nb, (slot + 3) % 4).wait()
        if not last:
            idx_wait(i + 1, nslot)
            gather_desc(i + 1, nb, nslot).start()
            ef_desc(i + 1, nb).start()
        scat_desc(i, b, slot).start(add=True)
        scat_ef_desc(i, b, slot).start(add=True)

    pltpu.sync_copy(sd3_hbm.at[wid, 0], ibuf.at[0])
    gather_desc(0, 0, 0).start()
    ef_desc(0, 0).start()
    step(0, 0, 0, first=True)

    def group(g, carry):
        i0 = 1 + 4 * g
        for p in range(4):
            step(i0 + p, (1 + p) % 2, (1 + p) % 4)
        return carry

    NG = (STEPS - 2) // 4
    lax.fori_loop(0, NG, group, 0)

    for i in range(1 + 4 * NG, STEPS):
        step(i, i % 2, i % 4, last=(i == STEPS - 1))
    scat_desc(STEPS - 1, (STEPS - 1) % 2, (STEPS - 1) % 4).wait()
    scat_ef_desc(STEPS - 1, (STEPS - 1) % 2, (STEPS - 1) % 4).wait()
    plsc.subcore_barrier()

    pltpu.sync_copy(acc.at[pl.ds(s * RPT, RPT)],
                    part_hbm.at[c, pl.ds(s * RPT, RPT)])
    @pl.when(s == NS - 1)
    def _():
        pltpu.sync_copy(acc.at[pl.ds(RPT * NS, TAIL)],
                        part_hbm.at[c, pl.ds(RPT * NS, TAIL)])


_sc_segment_sum = pl.kernel(
    _sc_body,
    out_type=jax.ShapeDtypeStruct((NC, N, D), jnp.float32),
    mesh=plsc.VectorSubcoreMesh(core_axis_name="c", subcore_axis_name="s"),
    scratch_types=[
        pltpu.VMEM((4, 2, K), jnp.int32),
        pltpu.VMEM((2, K, D), jnp.float32),
        pltpu.VMEM((2, K, D), jnp.float32),
        pltpu.VMEM_SHARED((N, D), jnp.float32),
        pltpu.SemaphoreType.DMA,
        pltpu.SemaphoreType.DMA,
        pltpu.SemaphoreType.DMA,
        pltpu.SemaphoreType.DMA,
        pltpu.SemaphoreType.DMA,
    ],
)


BLK = 1000


def _mlp_body(p0_ref, p1_ref, w1_ref, b1_ref, w2_ref, b2_ref, o_ref):
    x = p0_ref[...] + p1_ref[...]
    h = jnp.dot(x, w1_ref[...], preferred_element_type=jnp.float32)
    h = jnp.maximum(h + b1_ref[...], 0.0)
    o = jnp.dot(h, w2_ref[...], preferred_element_type=jnp.float32)
    o_ref[...] = o + b2_ref[...]


_mlp = pl.pallas_call(
    _mlp_body,
    grid=(N // BLK,),
    in_specs=[
        pl.BlockSpec((BLK, D), lambda i: (i, 0)),
        pl.BlockSpec((BLK, D), lambda i: (i, 0)),
        pl.BlockSpec((D, H), lambda i: (0, 0)),
        pl.BlockSpec((1, H), lambda i: (0, 0)),
        pl.BlockSpec((H, O), lambda i: (0, 0)),
        pl.BlockSpec((1, O), lambda i: (0, 0)),
    ],
    out_specs=pl.BlockSpec((BLK, O), lambda i: (i, 0)),
    out_shape=jax.ShapeDtypeStruct((N, O), jnp.float32),
)


@jax.jit
def kernel(node_feats, edge_feats, edge_index, W1, b1, W2, b2):
    sd3 = edge_index.reshape(2, NW, STEPS, K).transpose(1, 2, 0, 3)
    zeros = jnp.zeros((N, D), jnp.float32)
    parts = _sc_segment_sum(node_feats, edge_feats, sd3, zeros)
    return _mlp(parts[0], parts[1], W1, b1.reshape(1, H), W2, b2.reshape(1, O))


# speedup vs baseline: 1.0120x; 1.0120x over previous
import functools

import jax
import jax.numpy as jnp
from jax import lax
from jax.experimental import pallas as pl
from jax.experimental.pallas import tpu as pltpu
from jax.experimental.pallas import tpu_sc as plsc

N = 10000
E = 320000
D = 128
H = 128
O = 128

NC = 2
NS = 16
NW = NC * NS
EPW = E // NW
K = 80
STEPS = EPW // K
RPT = 624
TAIL = N - RPT * NS


def _sc_body(node_hbm, ef_hbm, sd3_hbm, zeros_hbm, part_hbm,
             ibuf, rows, ef, acc, gsem, esem, ssem, isem):
    c = lax.axis_index("c")
    s = lax.axis_index("s")
    wid = c * NS + s
    base = wid * EPW

    pltpu.sync_copy(zeros_hbm.at[pl.ds(s * RPT, RPT)],
                    acc.at[pl.ds(s * RPT, RPT)])
    @pl.when(s == NS - 1)
    def _():
        pltpu.sync_copy(zeros_hbm.at[pl.ds(RPT * NS, TAIL)],
                        acc.at[pl.ds(RPT * NS, TAIL)])
    plsc.subcore_barrier()

    def idx_start(i, slot):
        pltpu.async_copy(sd3_hbm.at[wid, i], ibuf.at[slot], isem)

    def idx_wait(i, slot):
        pltpu.make_async_copy(sd3_hbm.at[wid, i], ibuf.at[slot], isem).wait()

    def gather_desc(i, b, slot):
        return pltpu.make_async_copy(
            node_hbm.at[ibuf.at[slot, 0]], rows.at[b], gsem)

    def ef_desc(i, b):
        return pltpu.make_async_copy(
            ef_hbm.at[pl.ds(base + i * K, K)], ef.at[b], esem)

    def scat_desc(i, b, slot):
        return pltpu.make_async_copy(
            rows.at[b], acc.at[ibuf.at[slot, 1]], ssem)

    def adds(b):
        @plsc.parallel_loop(0, K, 1, unroll=2)
        def addrow(r):
            for j in range(D // 16):
                sl = pl.ds(j * 16, 16)
                rows[b, r, sl] = rows[b, r, sl] + ef[b, r, sl]

    def step(i, b, slot, first=False, last=False):
        nb = 1 - b
        nslot = (slot + 1) % 4
        if not last:
            idx_start(i + 1, nslot)
        gather_desc(i, b, slot).wait()
        ef_desc(i, b).wait()
        if not first:
            scat_desc(i - 1, nb, (slot + 3) % 4).wait()
        if not last:
            idx_wait(i + 1, nslot)
            gather_desc(i + 1, nb, nslot).start()
            ef_desc(i + 1, nb).start()
        adds(b)
        scat_desc(i, b, slot).start(add=True)

    pltpu.sync_copy(sd3_hbm.at[wid, 0], ibuf.at[0])
    gather_desc(0, 0, 0).start()
    ef_desc(0, 0).start()
    step(0, 0, 0, first=True)

    def group(g, carry):
        i0 = 1 + 4 * g
        for p in range(4):
            step(i0 + p, (1 + p) % 2, (1 + p) % 4)
        return carry

    NG = (STEPS - 2) // 4
    lax.fori_loop(0, NG, group, 0)

    for i in range(1 + 4 * NG, STEPS):
        step(i, i % 2, i % 4, last=(i == STEPS - 1))
    scat_desc(STEPS - 1, (STEPS - 1) % 2, (STEPS - 1) % 4).wait()
    plsc.subcore_barrier()

    pltpu.sync_copy(acc.at[pl.ds(s * RPT, RPT)],
                    part_hbm.at[c, pl.ds(s * RPT, RPT)])
    @pl.when(s == NS - 1)
    def _():
        pltpu.sync_copy(acc.at[pl.ds(RPT * NS, TAIL)],
                        part_hbm.at[c, pl.ds(RPT * NS, TAIL)])


_sc_segment_sum = pl.kernel(
    _sc_body,
    out_type=jax.ShapeDtypeStruct((NC, N, D), jnp.float32),
    mesh=plsc.VectorSubcoreMesh(core_axis_name="c", subcore_axis_name="s"),
    scratch_types=[
        pltpu.VMEM((4, 2, K), jnp.int32),
        pltpu.VMEM((2, K, D), jnp.float32),
        pltpu.VMEM((2, K, D), jnp.float32),
        pltpu.VMEM_SHARED((N, D), jnp.float32),
        pltpu.SemaphoreType.DMA,
        pltpu.SemaphoreType.DMA,
        pltpu.SemaphoreType.DMA,
        pltpu.SemaphoreType.DMA,
    ],
)


BLK = 1000


def _mlp_body(p0_ref, p1_ref, w1_ref, b1_ref, w2_ref, b2_ref, o_ref):
    x = p0_ref[...] + p1_ref[...]
    h = jnp.dot(x, w1_ref[...], preferred_element_type=jnp.float32)
    h = jnp.maximum(h + b1_ref[...], 0.0)
    o = jnp.dot(h, w2_ref[...], preferred_element_type=jnp.float32)
    o_ref[...] = o + b2_ref[...]


_mlp = pl.pallas_call(
    _mlp_body,
    grid=(N // BLK,),
    in_specs=[
        pl.BlockSpec((BLK, D), lambda i: (i, 0)),
        pl.BlockSpec((BLK, D), lambda i: (i, 0)),
        pl.BlockSpec((D, H), lambda i: (0, 0)),
        pl.BlockSpec((1, H), lambda i: (0, 0)),
        pl.BlockSpec((H, O), lambda i: (0, 0)),
        pl.BlockSpec((1, O), lambda i: (0, 0)),
    ],
    out_specs=pl.BlockSpec((BLK, O), lambda i: (i, 0)),
    out_shape=jax.ShapeDtypeStruct((N, O), jnp.float32),
)


@jax.jit
def kernel(node_feats, edge_feats, edge_index, W1, b1, W2, b2):
    sd3 = edge_index.reshape(2, NW, STEPS, K).transpose(1, 2, 0, 3)
    zeros = jnp.zeros((N, D), jnp.float32)
    parts = _sc_segment_sum(node_feats, edge_feats, sd3, zeros)
    return _mlp(parts[0], parts[1], W1, b1.reshape(1, H), W2, b2.reshape(1, O))


# speedup vs baseline: 1.0847x; 1.0719x over previous
import functools

import jax
import jax.numpy as jnp
from jax import lax
from jax.experimental import pallas as pl
from jax.experimental.pallas import tpu as pltpu
from jax.experimental.pallas import tpu_sc as plsc

N = 10000
E = 320000
D = 128
H = 128
O = 128

NC = 2
NS = 16
NW = NC * NS
EPW = E // NW
K = 80
STEPS = EPW // K
RPT = 624
TAIL = N - RPT * NS


def _sc_body(node_hbm, ef_hbm, sd4_hbm, zeros_hbm, part_hbm,
             ibuf_s, ibuf_d, rows, ef, acc, gsem, esem, ssem, isem_s, isem_d):
    c = lax.axis_index("c")
    s = lax.axis_index("s")
    wid = c * NS + s
    base = wid * EPW

    pltpu.sync_copy(zeros_hbm.at[pl.ds(s * RPT, RPT)],
                    acc.at[pl.ds(s * RPT, RPT)])
    @pl.when(s == NS - 1)
    def _():
        pltpu.sync_copy(zeros_hbm.at[pl.ds(RPT * NS, TAIL)],
                        acc.at[pl.ds(RPT * NS, TAIL)])
    plsc.subcore_barrier()

    def idx_start(i, slot):
        pltpu.async_copy(sd4_hbm.at[0, wid * STEPS + i], ibuf_s.at[slot], isem_s)
        pltpu.async_copy(sd4_hbm.at[1, wid * STEPS + i], ibuf_d.at[slot], isem_d)

    def idx_wait(i, slot):
        pltpu.make_async_copy(
            sd4_hbm.at[0, wid * STEPS + i], ibuf_s.at[slot], isem_s).wait()
        pltpu.make_async_copy(
            sd4_hbm.at[1, wid * STEPS + i], ibuf_d.at[slot], isem_d).wait()

    def gather_desc(i, b, slot):
        return pltpu.make_async_copy(
            node_hbm.at[ibuf_s.at[slot, 0]], rows.at[b], gsem)

    def ef_desc(i, b):
        return pltpu.make_async_copy(
            ef_hbm.at[pl.ds(base + i * K, K)], ef.at[b], esem)

    def scat_desc(i, b, slot):
        return pltpu.make_async_copy(
            rows.at[b], acc.at[ibuf_d.at[slot, 0]], ssem)

    def adds(b):
        def addrow(r, c2):
            for j in range(D // 16):
                sl = pl.ds(j * 16, 16)
                rows[b, r, sl] = rows[b, r, sl] + ef[b, r, sl]
            return c2
        lax.fori_loop(0, K, addrow, 0)

    def step(i, b, slot, first=False, last=False):
        nb = 1 - b
        nslot = (slot + 1) % 4
        if not last:
            idx_start(i + 1, nslot)
        gather_desc(i, b, slot).wait()
        ef_desc(i, b).wait()
        if not first:
            scat_desc(i - 1, nb, (slot + 3) % 4).wait()
        if not last:
            idx_wait(i + 1, nslot)
            gather_desc(i + 1, nb, nslot).start()
            ef_desc(i + 1, nb).start()
        adds(b)
        scat_desc(i, b, slot).start(add=True)

    pltpu.sync_copy(sd4_hbm.at[0, wid * STEPS], ibuf_s.at[0])
    pltpu.sync_copy(sd4_hbm.at[1, wid * STEPS], ibuf_d.at[0])
    gather_desc(0, 0, 0).start()
    ef_desc(0, 0).start()
    step(0, 0, 0, first=True)

    def group(g, carry):
        i0 = 1 + 4 * g
        for p in range(4):
            step(i0 + p, (1 + p) % 2, (1 + p) % 4)
        return carry

    NG = (STEPS - 2) // 4
    lax.fori_loop(0, NG, group, 0)

    for i in range(1 + 4 * NG, STEPS):
        step(i, i % 2, i % 4, last=(i == STEPS - 1))
    scat_desc(STEPS - 1, (STEPS - 1) % 2, (STEPS - 1) % 4).wait()
    plsc.subcore_barrier()

    pltpu.sync_copy(acc.at[pl.ds(s * RPT, RPT)],
                    part_hbm.at[c, pl.ds(s * RPT, RPT)])
    @pl.when(s == NS - 1)
    def _():
        pltpu.sync_copy(acc.at[pl.ds(RPT * NS, TAIL)],
                        part_hbm.at[c, pl.ds(RPT * NS, TAIL)])


_sc_segment_sum = pl.kernel(
    _sc_body,
    out_type=jax.ShapeDtypeStruct((NC, N, D), jnp.float32),
    mesh=plsc.VectorSubcoreMesh(core_axis_name="c", subcore_axis_name="s"),
    scratch_types=[
        pltpu.VMEM((4, 1, K), jnp.int32),
        pltpu.VMEM((4, 1, K), jnp.int32),
        pltpu.VMEM((2, K, D), jnp.float32),
        pltpu.VMEM((2, K, D), jnp.float32),
        pltpu.VMEM_SHARED((N, D), jnp.float32),
        pltpu.SemaphoreType.DMA,
        pltpu.SemaphoreType.DMA,
        pltpu.SemaphoreType.DMA,
        pltpu.SemaphoreType.DMA,
        pltpu.SemaphoreType.DMA,
    ],
)


BLK = 2000


def _mlp_body(p0_ref, p1_ref, w1_ref, b1_ref, w2_ref, b2_ref, o_ref):
    x = p0_ref[0] + p1_ref[0]
    h = jnp.dot(x, w1_ref[...], preferred_element_type=jnp.float32)
    h = jnp.maximum(h + b1_ref[...], 0.0)
    o = jnp.dot(h, w2_ref[...], preferred_element_type=jnp.float32)
    o_ref[...] = o + b2_ref[...]


_mlp = pl.pallas_call(
    _mlp_body,
    grid=(N // BLK,),
    in_specs=[
        pl.BlockSpec((1, BLK, D), lambda i: (0, i, 0)),
        pl.BlockSpec((1, BLK, D), lambda i: (1, i, 0)),
        pl.BlockSpec((D, H), lambda i: (0, 0)),
        pl.BlockSpec((1, H), lambda i: (0, 0)),
        pl.BlockSpec((H, O), lambda i: (0, 0)),
        pl.BlockSpec((1, O), lambda i: (0, 0)),
    ],
    out_specs=pl.BlockSpec((BLK, O), lambda i: (i, 0)),
    out_shape=jax.ShapeDtypeStruct((N, O), jnp.float32),
)


@jax.jit
def kernel(node_feats, edge_feats, edge_index, W1, b1, W2, b2):
    sd4 = edge_index.reshape(2, NW * STEPS, 1, K)
    zeros = jnp.zeros((N, D), jnp.float32)
    parts = _sc_segment_sum(node_feats, edge_feats, sd4, zeros)
    return _mlp(parts, parts, W1, b1.reshape(1, H), W2, b2.reshape(1, O))


# speedup vs baseline: 1.1071x; 1.0207x over previous
import functools

import jax
import jax.numpy as jnp
from jax import lax
from jax.experimental import pallas as pl
from jax.experimental.pallas import tpu as pltpu
from jax.experimental.pallas import tpu_sc as plsc

N = 10000
E = 320000
D = 128
H = 128
O = 128

NC = 2
NS = 16
NW = NC * NS
EPW = E // NW
K = 80
STEPS = EPW // K
RPT = 624
TAIL = N - RPT * NS


def _sc_body(node_hbm, ef_hbm, sd4_hbm, part_hbm,
             ibuf_s, ibuf_d, rows, ef, acc, gsem, esem, ssem, isem_s, isem_d):
    c = lax.axis_index("c")
    s = lax.axis_index("s")
    wid = c * NS + s
    base = wid * EPW

    def zrow(r, c2):
        for j in range(D // 16):
            rows[1, r, pl.ds(j * 16, 16)] = jnp.zeros((16,), jnp.float32)
        return c2
    lax.fori_loop(0, K, zrow, 0)
    for t in range(7):
        pltpu.sync_copy(rows.at[1], acc.at[pl.ds(s * RPT + t * K, K)])
    pltpu.sync_copy(rows.at[1, pl.ds(0, RPT - 7 * K)],
                    acc.at[pl.ds(s * RPT + 7 * K, RPT - 7 * K)])
    @pl.when(s == NS - 1)
    def _():
        pltpu.sync_copy(rows.at[1, pl.ds(0, TAIL)],
                        acc.at[pl.ds(RPT * NS, TAIL)])
    plsc.subcore_barrier()

    def idx_start(i, slot):
        pltpu.async_copy(sd4_hbm.at[0, wid * STEPS + i], ibuf_s.at[slot], isem_s)
        pltpu.async_copy(sd4_hbm.at[1, wid * STEPS + i], ibuf_d.at[slot], isem_d)

    def idx_wait(i, slot):
        pltpu.make_async_copy(
            sd4_hbm.at[0, wid * STEPS + i], ibuf_s.at[slot], isem_s).wait()
        pltpu.make_async_copy(
            sd4_hbm.at[1, wid * STEPS + i], ibuf_d.at[slot], isem_d).wait()

    def gather_desc(i, b, slot):
        return pltpu.make_async_copy(
            node_hbm.at[ibuf_s.at[slot, 0]], rows.at[b], gsem)

    def ef_desc(i, b):
        return pltpu.make_async_copy(
            ef_hbm.at[pl.ds(base + i * K, K)], ef.at[b], esem)

    def scat_desc(i, b, slot):
        return pltpu.make_async_copy(
            rows.at[b], acc.at[ibuf_d.at[slot, 0]], ssem)

    def adds(b):
        def addrow(r, c2):
            for j in range(D // 16):
                sl = pl.ds(j * 16, 16)
                rows[b, r, sl] = rows[b, r, sl] + ef[b, r, sl]
            return c2
        lax.fori_loop(0, K, addrow, 0)

    def step(i, b, slot, first=False, last=False):
        nb = 1 - b
        nslot = (slot + 1) % 4
        if not last:
            idx_start(i + 1, nslot)
        gather_desc(i, b, slot).wait()
        ef_desc(i, b).wait()
        if not first:
            scat_desc(i - 1, nb, (slot + 3) % 4).wait()
        if not last:
            idx_wait(i + 1, nslot)
            gather_desc(i + 1, nb, nslot).start()
            ef_desc(i + 1, nb).start()
        adds(b)
        scat_desc(i, b, slot).start(add=True)

    pltpu.sync_copy(sd4_hbm.at[0, wid * STEPS], ibuf_s.at[0])
    pltpu.sync_copy(sd4_hbm.at[1, wid * STEPS], ibuf_d.at[0])
    gather_desc(0, 0, 0).start()
    ef_desc(0, 0).start()
    step(0, 0, 0, first=True)

    def group(g, carry):
        i0 = 1 + 4 * g
        for p in range(4):
            step(i0 + p, (1 + p) % 2, (1 + p) % 4)
        return carry

    NG = (STEPS - 2) // 4
    lax.fori_loop(0, NG, group, 0)

    for i in range(1 + 4 * NG, STEPS):
        step(i, i % 2, i % 4, last=(i == STEPS - 1))
    scat_desc(STEPS - 1, (STEPS - 1) % 2, (STEPS - 1) % 4).wait()
    plsc.subcore_barrier()

    pltpu.sync_copy(acc.at[pl.ds(s * RPT, RPT)],
                    part_hbm.at[c, pl.ds(s * RPT, RPT)])
    @pl.when(s == NS - 1)
    def _():
        pltpu.sync_copy(acc.at[pl.ds(RPT * NS, TAIL)],
                        part_hbm.at[c, pl.ds(RPT * NS, TAIL)])


_sc_segment_sum = pl.kernel(
    _sc_body,
    out_type=jax.ShapeDtypeStruct((NC, N, D), jnp.float32),
    mesh=plsc.VectorSubcoreMesh(core_axis_name="c", subcore_axis_name="s"),
    scratch_types=[
        pltpu.VMEM((4, 1, K), jnp.int32),
        pltpu.VMEM((4, 1, K), jnp.int32),
        pltpu.VMEM((2, K, D), jnp.float32),
        pltpu.VMEM((2, K, D), jnp.float32),
        pltpu.VMEM_SHARED((N, D), jnp.float32),
        pltpu.SemaphoreType.DMA,
        pltpu.SemaphoreType.DMA,
        pltpu.SemaphoreType.DMA,
        pltpu.SemaphoreType.DMA,
        pltpu.SemaphoreType.DMA,
    ],
)


BLK = 2000


def _mlp_body(p0_ref, p1_ref, w1_ref, b1_ref, w2_ref, b2_ref, o_ref):
    x = p0_ref[0] + p1_ref[0]
    h = jnp.dot(x, w1_ref[...], preferred_element_type=jnp.float32)
    h = jnp.maximum(h + b1_ref[...], 0.0)
    o = jnp.dot(h, w2_ref[...], preferred_element_type=jnp.float32)
    o_ref[...] = o + b2_ref[...]


_mlp = pl.pallas_call(
    _mlp_body,
    grid=(N // BLK,),
    in_specs=[
        pl.BlockSpec((1, BLK, D), lambda i: (0, i, 0)),
        pl.BlockSpec((1, BLK, D), lambda i: (1, i, 0)),
        pl.BlockSpec((D, H), lambda i: (0, 0)),
        pl.BlockSpec((1, H), lambda i: (0, 0)),
        pl.BlockSpec((H, O), lambda i: (0, 0)),
        pl.BlockSpec((1, O), lambda i: (0, 0)),
    ],
    out_specs=pl.BlockSpec((BLK, O), lambda i: (i, 0)),
    out_shape=jax.ShapeDtypeStruct((N, O), jnp.float32),
)


@jax.jit
def kernel(node_feats, edge_feats, edge_index, W1, b1, W2, b2):
    sd4 = edge_index.reshape(2, NW * STEPS, 1, K)
    parts = _sc_segment_sum(node_feats, edge_feats, sd4)
    return _mlp(parts, parts, W1, b1.reshape(1, H), W2, b2.reshape(1, O))
